# baseline (speedup 1.0000x reference)
import functools

import jax
import jax.numpy as jnp
from jax import lax
from jax.experimental import pallas as pl
from jax.experimental.pallas import tpu as pltpu
from jax.experimental.pallas import tpu_sc as plsc

_NC = 2
_NS = 16
_NW = _NC * _NS
_LANES = 16
_C = 8


def kernel(x, emb_table):
    B, T, D = x.shape
    rows_per_w = T // _NW
    n_chunks = rows_per_w // _C

    x_flat = x.reshape(B * T, D)
    emb_flat = emb_table[:T]

    mesh = plsc.VectorSubcoreMesh(core_axis_name="c", subcore_axis_name="s")

    @functools.partial(
        pl.kernel,
        out_type=jax.ShapeDtypeStruct((B * T, D), jnp.float32),
        mesh=mesh,
        scratch_types=(
            [pltpu.VMEM((_C, D), jnp.float32)] * (2 * (B + 1))
            + [pltpu.SemaphoreType.DMA] * 4
        ),
    )
    def sc_add(x_hbm, emb_hbm, out_hbm, *scr):
        ebuf = [scr[0], scr[1]]
        xbuf = [scr[2 : 2 + B], scr[2 + B : 2 + 2 * B]]
        sem_in = [scr[2 + 2 * B], scr[3 + 2 * B]]
        sem_out = [scr[4 + 2 * B], scr[5 + 2 * B]]

        wid = lax.axis_index("s") * _NC + lax.axis_index("c")
        t0 = wid * rows_per_w

        def e_copy(tc, p):
            row = t0 + tc * _C
            return pltpu.make_async_copy(
                emb_hbm.at[pl.ds(row, _C)], ebuf[p], sem_in[p]
            )

        def x_copy(tc, p, b):
            row = t0 + tc * _C
            return pltpu.make_async_copy(
                x_hbm.at[pl.ds(b * T + row, _C)], xbuf[p][b], sem_in[p]
            )

        def o_copy(tc, p, b):
            row = t0 + tc * _C
            return pltpu.make_async_copy(
                xbuf[p][b], out_hbm.at[pl.ds(b * T + row, _C)], sem_out[p]
            )

        def issue_in(tc, p):
            e_copy(tc, p).start()
            for b in range(B):
                x_copy(tc, p, b).start()

        def wait_in(tc, p):
            e_copy(tc, p).wait()
            for b in range(B):
                x_copy(tc, p, b).wait()

        def compute(p):
            eb = ebuf[p]
            xb = xbuf[p]
            for r in range(_C):
                @plsc.parallel_loop(0, D // _LANES, unroll=8)
                def _(i):
                    off = i * _LANES
                    e = eb[r, pl.ds(off, _LANES)]
                    for b in range(B):
                        plsc.addupdate(xb[b].at[r, pl.ds(off, _LANES)], e)

        issue_in(0, 0)

        def outer(k, carry):
            for par in range(2):
                tc = 2 * k + par
                nxt = 1 - par

                @pl.when(tc + 1 < n_chunks)
                def _():
                    @pl.when(tc >= 1)
                    def _():
                        for b in range(B):
                            o_copy(tc - 1, nxt, b).wait()

                    issue_in(tc + 1, nxt)

                wait_in(tc, par)
                compute(par)
                for b in range(B):
                    o_copy(tc, par, b).start()
            return carry

        lax.fori_loop(0, n_chunks // 2, outer, 0)
        for b in range(B):
            o_copy(n_chunks - 2, 0, b).wait()
            o_copy(n_chunks - 1, 1, b).wait()

    out = sc_add(x_flat, emb_flat)
    return out.reshape(B, T, D)


# speedup vs baseline: 1.0098x; 1.0098x over previous
import functools

import jax
import jax.numpy as jnp
from jax import lax
from jax.experimental import pallas as pl
from jax.experimental.pallas import tpu as pltpu
from jax.experimental.pallas import tpu_sc as plsc

_NC = 2
_NS = 16
_NW = _NC * _NS
_LANES = 16
_C = 8


def kernel(x, emb_table):
    B, T, D = x.shape
    rows_per_w = T // _NW
    n_chunks = rows_per_w // _C

    x_flat = x.reshape(B * T, D)
    emb_flat = emb_table[:T]

    mesh = plsc.VectorSubcoreMesh(core_axis_name="c", subcore_axis_name="s")

    @functools.partial(
        pl.kernel,
        out_type=jax.ShapeDtypeStruct((B * T, D), jnp.float32),
        mesh=mesh,
        scratch_types=(
            [pltpu.VMEM((_C, D), jnp.float32)] * (2 * (B + 1))
            + [pltpu.SemaphoreType.DMA] * 4
        ),
    )
    def sc_add(x_hbm, emb_hbm, out_hbm, *scr):
        ebuf = [scr[0], scr[1]]
        xbuf = [scr[2 : 2 + B], scr[2 + B : 2 + 2 * B]]
        sem_in = [scr[2 + 2 * B], scr[3 + 2 * B]]
        sem_out = [scr[4 + 2 * B], scr[5 + 2 * B]]

        wid = lax.axis_index("s") * _NC + lax.axis_index("c")
        t0 = wid * rows_per_w

        def e_copy(tc, p):
            row = t0 + tc * _C
            return pltpu.make_async_copy(
                emb_hbm.at[pl.ds(row, _C)], ebuf[p], sem_in[p]
            )

        def x_copy(tc, p, b):
            row = t0 + tc * _C
            return pltpu.make_async_copy(
                x_hbm.at[pl.ds(b * T + row, _C)], xbuf[p][b], sem_in[p]
            )

        def o_copy(tc, p, b):
            row = t0 + tc * _C
            return pltpu.make_async_copy(
                xbuf[p][b], out_hbm.at[pl.ds(b * T + row, _C)], sem_out[p]
            )

        def issue_in(tc, p):
            e_copy(tc, p).start()
            for b in range(B):
                x_copy(tc, p, b).start()

        def wait_in(tc, p):
            e_copy(tc, p).wait()
            for b in range(B):
                x_copy(tc, p, b).wait()

        def compute_b(p, tc, b):
            eb = ebuf[p]
            xb = xbuf[p][b]

            @plsc.parallel_loop(0, _C * D // _LANES, unroll=8)
            def _(i):
                off = i * _LANES
                r = i // (D // _LANES)
                c = (i % (D // _LANES)) * _LANES
                plsc.addupdate(xb.at[r, pl.ds(c, _LANES)], eb[r, pl.ds(c, _LANES)])

        issue_in(0, 0)

        def outer(k, carry):
            for par in range(2):
                tc = 2 * k + par
                nxt = 1 - par

                @pl.when(tc + 1 < n_chunks)
                def _():
                    @pl.when(tc >= 1)
                    def _():
                        for b in range(B):
                            o_copy(tc - 1, nxt, b).wait()

                    issue_in(tc + 1, nxt)

                wait_in(tc, par)
                for b in range(B):
                    compute_b(par, tc, b)
                    o_copy(tc, par, b).start()
            return carry

        lax.fori_loop(0, n_chunks // 2, outer, 0)
        for b in range(B):
            o_copy(n_chunks - 2, 0, b).wait()
            o_copy(n_chunks - 1, 1, b).wait()

    out = sc_add(x_flat, emb_flat)
    return out.reshape(B, T, D)


# speedup vs baseline: 1.0167x; 1.0069x over previous
import functools

import jax
import jax.numpy as jnp
from jax import lax
from jax.experimental import pallas as pl
from jax.experimental.pallas import tpu as pltpu
from jax.experimental.pallas import tpu_sc as plsc

_NC = 2
_NS = 16
_NW = _NC * _NS
_LANES = 16
_C = 4
_SETS = 4


def kernel(x, emb_table):
    B, T, D = x.shape
    rows_per_w = T // _NW
    n_chunks = rows_per_w // _C
    nv = _C * D // _LANES
    npb = D // _LANES

    x_flat = x.reshape(B * T, D)
    emb_flat = emb_table[:T]

    mesh = plsc.VectorSubcoreMesh(core_axis_name="c", subcore_axis_name="s")

    @functools.partial(
        pl.kernel,
        out_type=jax.ShapeDtypeStruct((B * T, D), jnp.float32),
        mesh=mesh,
        scratch_types=(
            [pltpu.VMEM((_C, D), jnp.float32)] * (_SETS * (B + 1))
            + [pltpu.SemaphoreType.DMA] * (2 * _SETS)
        ),
    )
    def sc_add(x_hbm, emb_hbm, out_hbm, *scr):
        nb = _SETS * (B + 1)
        ebuf = [scr[s * (B + 1)] for s in range(_SETS)]
        xbuf = [scr[s * (B + 1) + 1 : (s + 1) * (B + 1)] for s in range(_SETS)]
        sem_in = list(scr[nb : nb + _SETS])
        sem_out = list(scr[nb + _SETS : nb + 2 * _SETS])

        wid = lax.axis_index("s") * _NC + lax.axis_index("c")
        t0 = wid * rows_per_w

        def e_copy(tc, p):
            row = t0 + tc * _C
            return pltpu.make_async_copy(
                emb_hbm.at[pl.ds(row, _C)], ebuf[p], sem_in[p]
            )

        def x_copy(tc, p, b):
            row = t0 + tc * _C
            return pltpu.make_async_copy(
                x_hbm.at[pl.ds(b * T + row, _C)], xbuf[p][b], sem_in[p]
            )

        def o_copy(tc, p, b):
            row = t0 + tc * _C
            return pltpu.make_async_copy(
                xbuf[p][b], out_hbm.at[pl.ds(b * T + row, _C)], sem_out[p]
            )

        def issue_in(tc, p):
            e_copy(tc, p).start()
            for b in range(B):
                x_copy(tc, p, b).start()

        def wait_in(tc, p):
            e_copy(tc, p).wait()
            for b in range(B):
                x_copy(tc, p, b).wait()

        def compute_b(p, b):
            eb = ebuf[p]
            xb = xbuf[p][b]

            @plsc.parallel_loop(0, nv, unroll=8)
            def _(i):
                r = i // npb
                c = (i % npb) * _LANES
                plsc.addupdate(xb.at[r, pl.ds(c, _LANES)], eb[r, pl.ds(c, _LANES)])

        issue_in(0, 0)
        issue_in(1, 1)

        def outer(k, carry):
            for par in range(_SETS):
                tc = _SETS * k + par

                @pl.when(tc + 2 < n_chunks)
                def _():
                    nxt = (par + 2) % _SETS

                    @pl.when(tc >= 2)
                    def _():
                        for b in range(B):
                            o_copy(tc - 2, nxt, b).wait()

                    issue_in(tc + 2, nxt)

                wait_in(tc, par)
                for b in range(B):
                    compute_b(par, b)
                    o_copy(tc, par, b).start()
            return carry

        lax.fori_loop(0, n_chunks // _SETS, outer, 0)
        for tc in range(n_chunks - 4, n_chunks):
            for b in range(B):
                o_copy(tc, tc % _SETS, b).wait()

    out = sc_add(x_flat, emb_flat)
    return out.reshape(B, T, D)
